# baseline (speedup 1.0000x reference)
import jax
import jax.numpy as jnp
from jax.experimental import pallas as pl

N = 100000
D_X = 128
D_E = 12
D_OUT = D_E + D_X
TC_BLOCK = 2000
NB = N // TC_BLOCK


def kernel(residue, x, embed_weight):
    table = jnp.zeros((24, 16), jnp.float32).at[:20, :D_E].set(embed_weight)
    res3 = residue.reshape(NB, 1, TC_BLOCK)

    def body(res_ref, tab_ref, x_ref, o_ref):
        res = res_ref[0, 0, :]
        onehot = (res[:, None] == jax.lax.broadcasted_iota(
            jnp.int32, (1, 24), 1)).astype(jnp.float32)
        emb = jnp.dot(onehot, tab_ref[...],
                      preferred_element_type=jnp.float32)
        o_ref[...] = jnp.concatenate([emb[:, :D_E], x_ref[...]], axis=1)

    return pl.pallas_call(
        body,
        grid=(NB,),
        in_specs=[
            pl.BlockSpec((1, 1, TC_BLOCK), lambda i: (i, 0, 0)),
            pl.BlockSpec((24, 16), lambda i: (0, 0)),
            pl.BlockSpec((TC_BLOCK, D_X), lambda i: (i, 0)),
        ],
        out_specs=pl.BlockSpec((TC_BLOCK, D_OUT), lambda i: (i, 0)),
        out_shape=jax.ShapeDtypeStruct((N, D_OUT), jnp.float32),
    )(res3, table, x)
